# Initial kernel scaffold; baseline (speedup 1.0000x reference)
import jax
import jax.numpy as jnp
from jax.experimental import pallas as pl


def kernel(feature, edge_index, subgraph_idx, norm, W):
    raise NotImplementedError("write your pallas kernel here")


# speedup vs baseline: 3.4521x; 3.4521x over previous
import functools

import jax
import jax.numpy as jnp
from jax import lax
from jax.experimental import pallas as pl
from jax.experimental.pallas import tpu as pltpu
from jax.experimental.pallas import tpu_sc as plsc

N = 10000
E = 320000
D = 128
NDIV = 9
ROWS = NDIV * N

NC = 2
NS = 16
G = 128
E_PAD = 327680
EPT = E_PAD // NS
NG = EPT // G
CH = 11264
NPASS = 4
SP_ROWS = CH + 128
ACC_ROWS = NC * NPASS * CH
PAD_DST = ROWS

BN = 400


def _whall_body(f_ref, n_ref, w_ref, o_ref):
    w = w_ref[0]
    acc = lax.dot_general(f_ref[...], w, (((1,), (1,)), ((), ())),
                          preferred_element_type=jnp.float32)
    o_ref[0] = acc * n_ref[...]


def _whall(feature, norm, W):
    return pl.pallas_call(
        _whall_body,
        grid=(NDIV, N // BN),
        in_specs=[
            pl.BlockSpec((BN, D), lambda d, i: (i, 0)),
            pl.BlockSpec((BN, 1), lambda d, i: (i, 0)),
            pl.BlockSpec((1, D, D), lambda d, i: (d, 0, 0)),
        ],
        out_specs=pl.BlockSpec((1, BN, D), lambda d, i: (d, i, 0)),
        out_shape=jax.ShapeDtypeStruct((NDIV, N, D), jnp.float32),
    )(feature, norm, W)


def _idx_body(src_ref, dst_ref, div_ref, o_ref):
    i = pl.program_id(0)
    r = lax.broadcasted_iota(jnp.int32, src_ref.shape, 0)
    c = lax.broadcasted_iota(jnp.int32, src_ref.shape, 1)
    pos = (i * src_ref.shape[0] + r) * G + c
    valid = pos < E
    div = div_ref[...]
    o_ref[:, 0, :] = jnp.where(valid, div * N + src_ref[...], 0)
    o_ref[:, 1, :] = jnp.where(valid, div * N + dst_ref[...], PAD_DST)


def _edge_indices(src_p, dst_p, div_p):
    NGRP = E_PAD // G
    RB = 64
    spec = pl.BlockSpec((RB, G), lambda i: (i, 0))
    return pl.pallas_call(
        _idx_body,
        grid=(NGRP // RB,),
        in_specs=[spec, spec, spec],
        out_specs=pl.BlockSpec((RB, 2, G), lambda i: (i, 0, 0)),
        out_shape=jax.ShapeDtypeStruct((NGRP, 2, G), jnp.int32),
    )(src_p, dst_p, div_p)


def _issue_idx_fetch(tcat_hbm, idx_v, s, g, buf, sem):
    pltpu.make_async_copy(tcat_hbm.at[s * NG + g], idx_v.at[buf], sem).start()


def _sc_body(whall_hbm, tcat_hbm, acc_hbm,
             idx_v, sidx_v, rows_v, zbuf, spacc,
             sem_i0, sem_i1, sem_r0, sem_r1):
    c = lax.axis_index("c")
    s = lax.axis_index("s")
    sem_i = (sem_i0, sem_i1)
    sem_r = (sem_r0, sem_r1)

    zv = jnp.zeros((16,), jnp.float32)

    def _zrow(r, carry):
        for k in range(8):
            zbuf[r, pl.ds(16 * k, 16)] = zv
        return carry

    lax.fori_loop(0, zbuf.shape[0], _zrow, 0)

    ZR = zbuf.shape[0]
    PT = SP_ROWS // NS
    WT = CH // NS

    for p in range(NPASS):
        base = (NC * p + c) * CH

        z0 = s * PT
        for j in range(PT // ZR):
            pltpu.sync_copy(zbuf.at[pl.ds(0, ZR)],
                            spacc.at[pl.ds(z0 + j * ZR, ZR)])
        if PT % ZR:
            pltpu.sync_copy(zbuf.at[pl.ds(0, PT % ZR)],
                            spacc.at[pl.ds(z0 + (PT // ZR) * ZR, PT % ZR)])
        plsc.subcore_barrier()

        _issue_idx_fetch(tcat_hbm, idx_v, s, 0, 0, sem_i[0])
        pltpu.make_async_copy(tcat_hbm.at[s * NG], idx_v.at[0],
                              sem_i[0]).wait()
        pltpu.make_async_copy(whall_hbm.at[idx_v.at[0].at[0]], rows_v.at[0],
                              sem_r[0]).start()
        _issue_idx_fetch(tcat_hbm, idx_v, s, 1, 1, sem_i[1])

        def _outer(o, carry):
            for b in range(2):
                g = o * 2 + b
                nb = 1 - b
                for k in range(8):
                    t = idx_v[b, 1, pl.ds(16 * k, 16)]
                    loc = t - base
                    ok = (loc >= 0) & (loc < CH)
                    sidx_v[b, pl.ds(16 * k, 16)] = jnp.where(ok, loc, CH)

                @pl.when(g + 1 < NG)
                def _():
                    pltpu.make_async_copy(tcat_hbm.at[s * NG + g + 1],
                                          idx_v.at[nb], sem_i[nb]).wait()
                    pltpu.make_async_copy(whall_hbm.at[idx_v.at[nb].at[0]],
                                          rows_v.at[nb], sem_r[nb]).start()

                pltpu.make_async_copy(whall_hbm.at[idx_v.at[b].at[0]],
                                      rows_v.at[b], sem_r[b]).wait()
                pltpu.sync_copy(rows_v.at[b], spacc.at[sidx_v.at[b]],
                                add=True)

                @pl.when(g + 2 < NG)
                def _():
                    pltpu.make_async_copy(tcat_hbm.at[s * NG + g + 2],
                                          idx_v.at[b], sem_i[b]).start()
            return carry

        lax.fori_loop(0, NG // 2, _outer, 0)
        plsc.subcore_barrier()

        w0 = s * WT
        pltpu.sync_copy(spacc.at[pl.ds(w0, WT)],
                        acc_hbm.at[pl.ds(base + w0, WT)])
        plsc.subcore_barrier()


def _sc_scatter(whall2d, tcat):
    mesh = plsc.VectorSubcoreMesh(core_axis_name="c", subcore_axis_name="s",
                                  num_cores=NC, num_subcores=NS)
    k = functools.partial(
        pl.kernel,
        out_type=jax.ShapeDtypeStruct((ACC_ROWS, D), jnp.float32),
        mesh=mesh,
        scratch_types=[
            pltpu.VMEM((2, 2, G), jnp.int32),
            pltpu.VMEM((2, G), jnp.int32),
            pltpu.VMEM((2, G, D), jnp.float32),
            pltpu.VMEM((32, D), jnp.float32),
            pltpu.VMEM_SHARED((SP_ROWS, D), jnp.float32),
            pltpu.SemaphoreType.DMA,
            pltpu.SemaphoreType.DMA,
            pltpu.SemaphoreType.DMA,
            pltpu.SemaphoreType.DMA,
        ],
    )(_sc_body)
    return k(whall2d, tcat)


def _epi_body(a_ref, n_ref, o_ref):
    o_ref[...] = jnp.maximum(a_ref[...] * n_ref[...], 0.0)


def _epilogue(acc, norm):
    return pl.pallas_call(
        _epi_body,
        grid=(N // BN, NDIV),
        in_specs=[
            pl.BlockSpec((BN, D), lambda i, d: (d * (N // BN) + i, 0)),
            pl.BlockSpec((BN, 1), lambda i, d: (i, 0)),
        ],
        out_specs=pl.BlockSpec((BN, D), lambda i, d: (i, d)),
        out_shape=jax.ShapeDtypeStruct((N, NDIV * D), jnp.float32),
    )(acc, norm)


def kernel(feature, edge_index, subgraph_idx, norm, W):
    pad = E_PAD - E
    src_p = jnp.pad(edge_index[0], (0, pad)).reshape(E_PAD // G, G)
    dst_p = jnp.pad(edge_index[1], (0, pad)).reshape(E_PAD // G, G)
    div_p = jnp.pad(subgraph_idx, (0, pad)).reshape(E_PAD // G, G)

    whall = _whall(feature, norm, W).reshape(ROWS, D)
    tcat = _edge_indices(src_p, dst_p, div_p)
    acc = _sc_scatter(whall, tcat)
    return _epilogue(acc, norm)


# speedup vs baseline: 9.9420x; 2.8800x over previous
import functools

import jax
import jax.numpy as jnp
from jax import lax
from jax.experimental import pallas as pl
from jax.experimental.pallas import tpu as pltpu
from jax.experimental.pallas import tpu_sc as plsc

N = 10000
E = 320000
D = 128
NDIV = 9
ROWS = NDIV * N

NC = 2
NS = 16
G = 128
E_PAD = 327680
EPT = E_PAD // NS
NG = EPT // G
CH = 11264
NPASS = 4
SP_ROWS = CH + 128
ACC_ROWS = NC * NPASS * CH
PAD_DST = ROWS

BN = 400


def _whall_body(f_ref, n_ref, w_ref, o_ref):
    w = w_ref[0]
    acc = lax.dot_general(f_ref[...], w, (((1,), (1,)), ((), ())),
                          preferred_element_type=jnp.float32)
    o_ref[0] = acc * n_ref[...]


def _whall(feature, norm, W):
    return pl.pallas_call(
        _whall_body,
        grid=(NDIV, N // BN),
        in_specs=[
            pl.BlockSpec((BN, D), lambda d, i: (i, 0)),
            pl.BlockSpec((BN, 1), lambda d, i: (i, 0)),
            pl.BlockSpec((1, D, D), lambda d, i: (d, 0, 0)),
        ],
        out_specs=pl.BlockSpec((1, BN, D), lambda d, i: (d, i, 0)),
        out_shape=jax.ShapeDtypeStruct((NDIV, N, D), jnp.float32),
    )(feature, norm, W)


def _idx_body(src_ref, dst_ref, div_ref, o_ref):
    i = pl.program_id(0)
    r = lax.broadcasted_iota(jnp.int32, src_ref.shape, 0)
    c = lax.broadcasted_iota(jnp.int32, src_ref.shape, 1)
    pos = (i * src_ref.shape[0] + r) * G + c
    valid = pos < E
    div = div_ref[...]
    o_ref[:, 0, :] = jnp.where(valid, div * N + src_ref[...], 0)
    o_ref[:, 1, :] = jnp.where(valid, div * N + dst_ref[...], PAD_DST)


def _edge_indices(src_p, dst_p, div_p):
    NGRP = E_PAD // G
    RB = 64
    spec = pl.BlockSpec((RB, G), lambda i: (i, 0))
    return pl.pallas_call(
        _idx_body,
        grid=(NGRP // RB,),
        in_specs=[spec, spec, spec],
        out_specs=pl.BlockSpec((RB, 2, G), lambda i: (i, 0, 0)),
        out_shape=jax.ShapeDtypeStruct((NGRP, 2, G), jnp.int32),
    )(src_p, dst_p, div_p)


RING = 4 * G


def _sc_body(whall_hbm, tcat_hbm, acc_hbm,
             idx_v, sidx2, rows_v, gstage, sstage, zbuf, spacc,
             sem_i0, sem_i1, sem_r0, sem_r1):
    c = lax.axis_index("c")
    s = lax.axis_index("s")
    sem_i = (sem_i0, sem_i1)
    sem_r = (sem_r0, sem_r1)

    zv = jnp.zeros((16,), jnp.float32)

    def _zrow(r, carry):
        for k in range(8):
            zbuf[r, pl.ds(16 * k, 16)] = zv
        return carry

    lax.fori_loop(0, zbuf.shape[0], _zrow, 0)

    ZR = zbuf.shape[0]
    PT = SP_ROWS // NS
    WT = CH // NS

    def _consume(j):
        poff = pl.multiple_of((j & 3) * G, G)
        for pp in range(2):
            @pl.when((j & 1) == pp)
            def _():
                for k in range(8):
                    sidx2[pp, pl.ds(16 * k, 16)] = \
                        sstage[pl.ds(poff + 16 * k, 16)]
                pltpu.make_async_copy(whall_hbm.at[gstage.at[pl.ds(0, G)]],
                                      rows_v.at[pp], sem_r[pp]).wait()
                pltpu.sync_copy(rows_v.at[pp], spacc.at[sidx2.at[pp]],
                                add=True)

    def _fire(j):
        off = pl.multiple_of((j & 3) * G, G)
        for pp in range(2):
            @pl.when((j & 1) == pp)
            def _():
                pltpu.make_async_copy(whall_hbm.at[gstage.at[pl.ds(off, G)]],
                                      rows_v.at[pp], sem_r[pp]).start()

        @pl.when(j >= 1)
        def _():
            _consume(j - 1)

    for p in range(NPASS):
        base = (NC * p + c) * CH

        z0 = s * PT
        for j in range(PT // ZR):
            pltpu.sync_copy(zbuf.at[pl.ds(0, ZR)],
                            spacc.at[pl.ds(z0 + j * ZR, ZR)])
        if PT % ZR:
            pltpu.sync_copy(zbuf.at[pl.ds(0, PT % ZR)],
                            spacc.at[pl.ds(z0 + (PT // ZR) * ZR, PT % ZR)])
        plsc.subcore_barrier()

        pltpu.make_async_copy(tcat_hbm.at[s * NG], idx_v.at[0],
                              sem_i[0]).start()
        pltpu.make_async_copy(tcat_hbm.at[s * NG + 1], idx_v.at[1],
                              sem_i[1]).start()

        def _outer(o, carry):
            cnt, nf = carry
            for b in range(2):
                g = o * 2 + b
                pltpu.make_async_copy(tcat_hbm.at[s * NG + g], idx_v.at[b],
                                      sem_i[b]).wait()
                for k in range(8):
                    ts = idx_v[b, 0, pl.ds(16 * k, 16)]
                    td = idx_v[b, 1, pl.ds(16 * k, 16)]
                    loc = td - base
                    ok = (loc >= 0) & (loc < CH)
                    oki = ok.astype(jnp.int32)
                    pos = (cnt + plsc.cumsum(oki) - 1) & (RING - 1)
                    plsc.store_scatter(gstage, [pos], ts, mask=ok)
                    plsc.store_scatter(sstage, [pos], loc, mask=ok)
                    cnt = cnt + jnp.sum(oki)
                @pl.when(g + 2 < NG)
                def _():
                    pltpu.make_async_copy(tcat_hbm.at[s * NG + g + 2],
                                          idx_v.at[b], sem_i[b]).start()
                fire_cond = (cnt - nf * G) >= G

                @pl.when(fire_cond)
                def _():
                    _fire(nf)
                nf = jnp.where(fire_cond, nf + 1, nf)
            return (cnt, nf)

        cnt, nf = lax.fori_loop(0, NG // 2, _outer,
                                (jnp.int32(0), jnp.int32(0)))

        lane = lax.broadcasted_iota(jnp.int32, (16,), 0)
        zsrc = jnp.zeros((16,), jnp.int32)
        zdst = jnp.full((16,), CH, jnp.int32)
        for k in range(8):
            pos = (cnt + 16 * k + lane) & (RING - 1)
            plsc.store_scatter(gstage, [pos], zsrc)
            plsc.store_scatter(sstage, [pos], zdst)
        nb_end = (cnt + G - 1) >> 7
        for _extra in range(2):
            fire_cond = nf < nb_end

            @pl.when(fire_cond)
            def _():
                _fire(nf)
            nf = jnp.where(fire_cond, nf + 1, nf)

        @pl.when(nf >= 1)
        def _():
            _consume(nf - 1)

        plsc.subcore_barrier()

        w0 = s * WT
        pltpu.sync_copy(spacc.at[pl.ds(w0, WT)],
                        acc_hbm.at[pl.ds(base + w0, WT)])
        plsc.subcore_barrier()


def _sc_scatter(whall2d, tcat):
    mesh = plsc.VectorSubcoreMesh(core_axis_name="c", subcore_axis_name="s",
                                  num_cores=NC, num_subcores=NS)
    k = functools.partial(
        pl.kernel,
        out_type=jax.ShapeDtypeStruct((ACC_ROWS, D), jnp.float32),
        mesh=mesh,
        compiler_params=pltpu.CompilerParams(needs_layout_passes=False),
        scratch_types=[
            pltpu.VMEM((2, 2, G), jnp.int32),
            pltpu.VMEM((2, G), jnp.int32),
            pltpu.VMEM((2, G, D), jnp.float32),
            pltpu.VMEM((RING,), jnp.int32),
            pltpu.VMEM((RING,), jnp.int32),
            pltpu.VMEM((16, D), jnp.float32),
            pltpu.VMEM_SHARED((SP_ROWS, D), jnp.float32),
            pltpu.SemaphoreType.DMA,
            pltpu.SemaphoreType.DMA,
            pltpu.SemaphoreType.DMA,
            pltpu.SemaphoreType.DMA,
        ],
    )(_sc_body)
    return k(whall2d, tcat)


def _epi_body(a_ref, n_ref, o_ref):
    o_ref[...] = jnp.maximum(a_ref[...] * n_ref[...], 0.0)


def _epilogue(acc, norm):
    return pl.pallas_call(
        _epi_body,
        grid=(N // BN, NDIV),
        in_specs=[
            pl.BlockSpec((BN, D), lambda i, d: (d * (N // BN) + i, 0)),
            pl.BlockSpec((BN, 1), lambda i, d: (i, 0)),
        ],
        out_specs=pl.BlockSpec((BN, D), lambda i, d: (i, d)),
        out_shape=jax.ShapeDtypeStruct((N, NDIV * D), jnp.float32),
    )(acc, norm)


def kernel(feature, edge_index, subgraph_idx, norm, W):
    pad = E_PAD - E
    src_p = jnp.pad(edge_index[0], (0, pad)).reshape(E_PAD // G, G)
    dst_p = jnp.pad(edge_index[1], (0, pad)).reshape(E_PAD // G, G)
    div_p = jnp.pad(subgraph_idx, (0, pad)).reshape(E_PAD // G, G)

    whall = _whall(feature, norm, W).reshape(ROWS, D)
    tcat = _edge_indices(src_p, dst_p, div_p)
    acc = _sc_scatter(whall, tcat)
    return _epilogue(acc, norm)


# speedup vs baseline: 10.4295x; 1.0490x over previous
import functools

import jax
import jax.numpy as jnp
from jax import lax
from jax.experimental import pallas as pl
from jax.experimental.pallas import tpu as pltpu
from jax.experimental.pallas import tpu_sc as plsc

N = 10000
E = 320000
D = 128
NDIV = 9
ROWS = NDIV * N

NC = 2
NS = 16
G = 128
E_PAD = 327680
EPT = E_PAD // NS
NG = EPT // G
CH = 11264
NPASS = 4
SP_ROWS = CH + 128
ACC_ROWS = NC * NPASS * CH
PAD_DST = ROWS

BN = 400


def _whall_body(f_ref, n_ref, w_ref, o_ref):
    w = w_ref[0]
    acc = lax.dot_general(f_ref[...], w, (((1,), (1,)), ((), ())),
                          preferred_element_type=jnp.float32)
    o_ref[0] = acc * n_ref[...]


def _whall(feature, norm, W):
    return pl.pallas_call(
        _whall_body,
        grid=(NDIV, N // BN),
        in_specs=[
            pl.BlockSpec((BN, D), lambda d, i: (i, 0)),
            pl.BlockSpec((BN, 1), lambda d, i: (i, 0)),
            pl.BlockSpec((1, D, D), lambda d, i: (d, 0, 0)),
        ],
        out_specs=pl.BlockSpec((1, BN, D), lambda d, i: (d, i, 0)),
        out_shape=jax.ShapeDtypeStruct((NDIV, N, D), jnp.float32),
    )(feature, norm, W)


def _idx_body(src_ref, dst_ref, div_ref, o_ref):
    i = pl.program_id(0)
    r = lax.broadcasted_iota(jnp.int32, src_ref.shape, 0)
    c = lax.broadcasted_iota(jnp.int32, src_ref.shape, 1)
    pos = (i * src_ref.shape[0] + r) * G + c
    valid = pos < E
    div = div_ref[...]
    o_ref[:, 0, :] = jnp.where(valid, div * N + src_ref[...], 0)
    o_ref[:, 1, :] = jnp.where(valid, div * N + dst_ref[...], PAD_DST)


def _edge_indices(src_p, dst_p, div_p):
    NGRP = E_PAD // G
    RB = 64
    spec = pl.BlockSpec((RB, G), lambda i: (i, 0))
    return pl.pallas_call(
        _idx_body,
        grid=(NGRP // RB,),
        in_specs=[spec, spec, spec],
        out_specs=pl.BlockSpec((RB, 2, G), lambda i: (i, 0, 0)),
        out_shape=jax.ShapeDtypeStruct((NGRP, 2, G), jnp.int32),
    )(src_p, dst_p, div_p)


RING = 4 * G
IB = 4
NB = NG // IB


def _sc_body(whall_hbm, tcat_hbm, acc_hbm,
             idx_v, sidx2, rows_v, gstage, sstage, zbuf, spacc,
             sem_i0, sem_i1, sem_r0, sem_r1):
    c = lax.axis_index("c")
    s = lax.axis_index("s")
    sem_i = (sem_i0, sem_i1)
    sem_r = (sem_r0, sem_r1)

    zv = jnp.zeros((16,), jnp.float32)

    def _zrow(r, carry):
        for k in range(8):
            zbuf[r, pl.ds(16 * k, 16)] = zv
        return carry

    lax.fori_loop(0, zbuf.shape[0], _zrow, 0)

    ZR = zbuf.shape[0]
    PT = SP_ROWS // NS
    WT = CH // NS

    def _consume(j):
        poff = pl.multiple_of((j & 3) * G, G)
        for pp in range(2):
            @pl.when((j & 1) == pp)
            def _():
                for k in range(8):
                    sidx2[pp, pl.ds(16 * k, 16)] = \
                        sstage[pl.ds(poff + 16 * k, 16)]
                pltpu.make_async_copy(whall_hbm.at[gstage.at[pl.ds(0, G)]],
                                      rows_v.at[pp], sem_r[pp]).wait()
                pltpu.sync_copy(rows_v.at[pp], spacc.at[sidx2.at[pp]],
                                add=True)

    def _fire(j):
        off = pl.multiple_of((j & 3) * G, G)
        for pp in range(2):
            @pl.when((j & 1) == pp)
            def _():
                pltpu.make_async_copy(whall_hbm.at[gstage.at[pl.ds(off, G)]],
                                      rows_v.at[pp], sem_r[pp]).start()

        @pl.when(j >= 1)
        def _():
            _consume(j - 1)

    for p in range(NPASS):
        base = (NC * p + c) * CH

        z0 = s * PT
        for j in range(PT // ZR):
            pltpu.sync_copy(zbuf.at[pl.ds(0, ZR)],
                            spacc.at[pl.ds(z0 + j * ZR, ZR)])
        if PT % ZR:
            pltpu.sync_copy(zbuf.at[pl.ds(0, PT % ZR)],
                            spacc.at[pl.ds(z0 + (PT // ZR) * ZR, PT % ZR)])
        plsc.subcore_barrier()

        pltpu.make_async_copy(tcat_hbm.at[pl.ds(s * NG, IB)], idx_v.at[0],
                              sem_i[0]).start()
        pltpu.make_async_copy(tcat_hbm.at[pl.ds(s * NG + IB, IB)],
                              idx_v.at[1], sem_i[1]).start()

        def _outer(o, carry):
            cntv, nf = carry
            for b in range(2):
                blk = o * 2 + b
                pltpu.make_async_copy(tcat_hbm.at[pl.ds(s * NG + blk * IB,
                                                        IB)],
                                      idx_v.at[b], sem_i[b]).wait()
                for j in range(IB):
                    for k in range(8):
                        ts = idx_v[b, j, 0, pl.ds(16 * k, 16)]
                        td = idx_v[b, j, 1, pl.ds(16 * k, 16)]
                        loc = td - base
                        ok = (loc >= 0) & (loc < CH)
                        oki = ok.astype(jnp.int32)
                        pos = (cntv + plsc.cumsum(oki) - 1) & (RING - 1)
                        plsc.store_scatter(gstage, [pos], ts, mask=ok)
                        plsc.store_scatter(sstage, [pos], loc, mask=ok)
                        cntv = cntv + plsc.all_reduce_population_count(ok)
                    fire_cond = (jnp.max(cntv) - nf * G) >= G

                    @pl.when(fire_cond)
                    def _():
                        _fire(nf)
                    nf = jnp.where(fire_cond, nf + 1, nf)
                @pl.when(blk + 2 < NB)
                def _():
                    pltpu.make_async_copy(
                        tcat_hbm.at[pl.ds(s * NG + (blk + 2) * IB, IB)],
                        idx_v.at[b], sem_i[b]).start()
            return (cntv, nf)

        cntv, nf = lax.fori_loop(0, NB // 2, _outer,
                                 (jnp.zeros((16,), jnp.int32), jnp.int32(0)))
        cnt = jnp.max(cntv)

        lane = lax.broadcasted_iota(jnp.int32, (16,), 0)
        zsrc = jnp.zeros((16,), jnp.int32)
        zdst = jnp.full((16,), CH, jnp.int32)
        for k in range(8):
            pos = (cnt + 16 * k + lane) & (RING - 1)
            plsc.store_scatter(gstage, [pos], zsrc)
            plsc.store_scatter(sstage, [pos], zdst)
        nb_end = (cnt + G - 1) >> 7
        for _extra in range(2):
            fire_cond = nf < nb_end

            @pl.when(fire_cond)
            def _():
                _fire(nf)
            nf = jnp.where(fire_cond, nf + 1, nf)

        @pl.when(nf >= 1)
        def _():
            _consume(nf - 1)

        plsc.subcore_barrier()

        w0 = s * WT
        pltpu.sync_copy(spacc.at[pl.ds(w0, WT)],
                        acc_hbm.at[pl.ds(base + w0, WT)])
        plsc.subcore_barrier()


def _sc_scatter(whall2d, tcat):
    mesh = plsc.VectorSubcoreMesh(core_axis_name="c", subcore_axis_name="s",
                                  num_cores=NC, num_subcores=NS)
    k = functools.partial(
        pl.kernel,
        out_type=jax.ShapeDtypeStruct((ACC_ROWS, D), jnp.float32),
        mesh=mesh,
        compiler_params=pltpu.CompilerParams(needs_layout_passes=False),
        scratch_types=[
            pltpu.VMEM((2, IB, 2, G), jnp.int32),
            pltpu.VMEM((2, G), jnp.int32),
            pltpu.VMEM((2, G, D), jnp.float32),
            pltpu.VMEM((RING,), jnp.int32),
            pltpu.VMEM((RING,), jnp.int32),
            pltpu.VMEM((16, D), jnp.float32),
            pltpu.VMEM_SHARED((SP_ROWS, D), jnp.float32),
            pltpu.SemaphoreType.DMA,
            pltpu.SemaphoreType.DMA,
            pltpu.SemaphoreType.DMA,
            pltpu.SemaphoreType.DMA,
        ],
    )(_sc_body)
    return k(whall2d, tcat)


def _epi_body(a_ref, n_ref, o_ref):
    o_ref[...] = jnp.maximum(a_ref[...] * n_ref[...], 0.0)


def _epilogue(acc, norm):
    return pl.pallas_call(
        _epi_body,
        grid=(N // BN, NDIV),
        in_specs=[
            pl.BlockSpec((BN, D), lambda i, d: (d * (N // BN) + i, 0)),
            pl.BlockSpec((BN, 1), lambda i, d: (i, 0)),
        ],
        out_specs=pl.BlockSpec((BN, D), lambda i, d: (i, d)),
        out_shape=jax.ShapeDtypeStruct((N, NDIV * D), jnp.float32),
    )(acc, norm)


def kernel(feature, edge_index, subgraph_idx, norm, W):
    pad = E_PAD - E
    src_p = jnp.pad(edge_index[0], (0, pad)).reshape(E_PAD // G, G)
    dst_p = jnp.pad(edge_index[1], (0, pad)).reshape(E_PAD // G, G)
    div_p = jnp.pad(subgraph_idx, (0, pad)).reshape(E_PAD // G, G)

    whall = _whall(feature, norm, W).reshape(ROWS, D)
    tcat = _edge_indices(src_p, dst_p, div_p)
    acc = _sc_scatter(whall, tcat)
    return _epilogue(acc, norm)


# speedup vs baseline: 10.4939x; 1.0062x over previous
import functools

import jax
import jax.numpy as jnp
from jax import lax
from jax.experimental import pallas as pl
from jax.experimental.pallas import tpu as pltpu
from jax.experimental.pallas import tpu_sc as plsc

N = 10000
E = 320000
D = 128
NDIV = 9
ROWS = NDIV * N

NC = 2
NS = 16
G = 128
E_PAD = 327680
EPT = E_PAD // NS
NG = EPT // G
CH = 11264
NPASS = 4
SP_ROWS = CH + 128
ACC_ROWS = NC * NPASS * CH
PAD_DST = ROWS

BN = 400


def _whall_body(f_ref, n_ref, w_ref, o_ref):
    w = w_ref[0]
    acc = lax.dot_general(f_ref[...], w, (((1,), (1,)), ((), ())),
                          preferred_element_type=jnp.float32)
    o_ref[0] = acc * n_ref[...]


def _whall(feature, norm, W):
    return pl.pallas_call(
        _whall_body,
        grid=(NDIV, N // BN),
        in_specs=[
            pl.BlockSpec((BN, D), lambda d, i: (i, 0)),
            pl.BlockSpec((BN, 1), lambda d, i: (i, 0)),
            pl.BlockSpec((1, D, D), lambda d, i: (d, 0, 0)),
        ],
        out_specs=pl.BlockSpec((1, BN, D), lambda d, i: (d, i, 0)),
        out_shape=jax.ShapeDtypeStruct((NDIV, N, D), jnp.float32),
    )(feature, norm, W)


def _idx_body(src_ref, dst_ref, div_ref, o_ref):
    i = pl.program_id(0)
    r = lax.broadcasted_iota(jnp.int32, src_ref.shape, 0)
    c = lax.broadcasted_iota(jnp.int32, src_ref.shape, 1)
    pos = (i * src_ref.shape[0] + r) * G + c
    valid = pos < E
    div = div_ref[...]
    o_ref[:, 0, :] = jnp.where(valid, div * N + src_ref[...], 0)
    o_ref[:, 1, :] = jnp.where(valid, div * N + dst_ref[...], PAD_DST)


def _edge_indices(src_p, dst_p, div_p):
    NGRP = E_PAD // G
    RB = 64
    spec = pl.BlockSpec((RB, G), lambda i: (i, 0))
    return pl.pallas_call(
        _idx_body,
        grid=(NGRP // RB,),
        in_specs=[spec, spec, spec],
        out_specs=pl.BlockSpec((RB, 2, G), lambda i: (i, 0, 0)),
        out_shape=jax.ShapeDtypeStruct((NGRP, 2, G), jnp.int32),
    )(src_p, dst_p, div_p)


RING = 4 * G
IB = 4
NB = NG // IB


def _sc_body(whall_hbm, tcat_hbm, acc_hbm,
             idx_v, sidx2, rows_v, gstage, sstage, zbuf, spacc,
             sem_i0, sem_i1, sem_r0, sem_r1, sem_z):
    c = lax.axis_index("c")
    s = lax.axis_index("s")
    sem_i = (sem_i0, sem_i1)
    sem_r = (sem_r0, sem_r1)

    zv = jnp.zeros((16,), jnp.float32)

    def _zrow(r, carry):
        for k in range(8):
            zbuf[r, pl.ds(16 * k, 16)] = zv
        return carry

    lax.fori_loop(0, zbuf.shape[0], _zrow, 0)

    ZR = zbuf.shape[0]
    PT = SP_ROWS // NS
    WT = CH // NS

    def _consume(j):
        poff = pl.multiple_of((j & 3) * G, G)
        for pp in range(2):
            @pl.when((j & 1) == pp)
            def _():
                for k in range(8):
                    sidx2[pp, pl.ds(16 * k, 16)] = \
                        sstage[pl.ds(poff + 16 * k, 16)]
                pltpu.make_async_copy(whall_hbm.at[gstage.at[pl.ds(0, G)]],
                                      rows_v.at[pp], sem_r[pp]).wait()
                pltpu.sync_copy(rows_v.at[pp], spacc.at[sidx2.at[pp]],
                                add=True)

    def _fire(j):
        off = pl.multiple_of((j & 3) * G, G)
        for pp in range(2):
            @pl.when((j & 1) == pp)
            def _():
                pltpu.make_async_copy(whall_hbm.at[gstage.at[pl.ds(off, G)]],
                                      rows_v.at[pp], sem_r[pp]).start()

        @pl.when(j >= 1)
        def _():
            _consume(j - 1)

    for p in range(NPASS):
        base = (NC * p + c) * CH

        z0 = s * PT
        for j in range(PT // ZR):
            pltpu.make_async_copy(zbuf.at[pl.ds(0, ZR)],
                                  spacc.at[pl.ds(z0 + j * ZR, ZR)],
                                  sem_z).start()
        if PT % ZR:
            pltpu.make_async_copy(zbuf.at[pl.ds(0, PT % ZR)],
                                  spacc.at[pl.ds(z0 + (PT // ZR) * ZR,
                                                 PT % ZR)],
                                  sem_z).start()
        for j in range(PT // ZR):
            pltpu.make_async_copy(zbuf.at[pl.ds(0, ZR)],
                                  spacc.at[pl.ds(z0 + j * ZR, ZR)],
                                  sem_z).wait()
        if PT % ZR:
            pltpu.make_async_copy(zbuf.at[pl.ds(0, PT % ZR)],
                                  spacc.at[pl.ds(z0 + (PT // ZR) * ZR,
                                                 PT % ZR)],
                                  sem_z).wait()
        plsc.subcore_barrier()

        pltpu.make_async_copy(tcat_hbm.at[pl.ds(s * NG, IB)], idx_v.at[0],
                              sem_i[0]).start()
        pltpu.make_async_copy(tcat_hbm.at[pl.ds(s * NG + IB, IB)],
                              idx_v.at[1], sem_i[1]).start()

        def _outer(o, carry):
            cntv, nf = carry
            for b in range(2):
                blk = o * 2 + b
                pltpu.make_async_copy(tcat_hbm.at[pl.ds(s * NG + blk * IB,
                                                        IB)],
                                      idx_v.at[b], sem_i[b]).wait()
                for j in range(IB):
                    for k in range(8):
                        ts = idx_v[b, j, 0, pl.ds(16 * k, 16)]
                        td = idx_v[b, j, 1, pl.ds(16 * k, 16)]
                        loc = td - base
                        ok = (loc >= 0) & (loc < CH)
                        oki = ok.astype(jnp.int32)
                        pos = (cntv + plsc.cumsum(oki) - 1) & (RING - 1)
                        plsc.store_scatter(gstage, [pos], ts, mask=ok)
                        plsc.store_scatter(sstage, [pos], loc, mask=ok)
                        cntv = cntv + plsc.all_reduce_population_count(ok)
                    fire_cond = (jnp.max(cntv) - nf * G) >= G

                    @pl.when(fire_cond)
                    def _():
                        _fire(nf)
                    nf = jnp.where(fire_cond, nf + 1, nf)
                @pl.when(blk + 2 < NB)
                def _():
                    pltpu.make_async_copy(
                        tcat_hbm.at[pl.ds(s * NG + (blk + 2) * IB, IB)],
                        idx_v.at[b], sem_i[b]).start()
            return (cntv, nf)

        cntv, nf = lax.fori_loop(0, NB // 2, _outer,
                                 (jnp.zeros((16,), jnp.int32), jnp.int32(0)))
        cnt = jnp.max(cntv)

        lane = lax.broadcasted_iota(jnp.int32, (16,), 0)
        zsrc = jnp.zeros((16,), jnp.int32)
        zdst = jnp.full((16,), CH, jnp.int32)
        for k in range(8):
            pos = (cnt + 16 * k + lane) & (RING - 1)
            plsc.store_scatter(gstage, [pos], zsrc)
            plsc.store_scatter(sstage, [pos], zdst)
        nb_end = (cnt + G - 1) >> 7
        for _extra in range(2):
            fire_cond = nf < nb_end

            @pl.when(fire_cond)
            def _():
                _fire(nf)
            nf = jnp.where(fire_cond, nf + 1, nf)

        @pl.when(nf >= 1)
        def _():
            _consume(nf - 1)

        plsc.subcore_barrier()

        w0 = s * WT
        pltpu.sync_copy(spacc.at[pl.ds(w0, WT)],
                        acc_hbm.at[pl.ds(base + w0, WT)])
        plsc.subcore_barrier()


def _sc_scatter(whall2d, tcat):
    mesh = plsc.VectorSubcoreMesh(core_axis_name="c", subcore_axis_name="s",
                                  num_cores=NC, num_subcores=NS)
    k = functools.partial(
        pl.kernel,
        out_type=jax.ShapeDtypeStruct((ACC_ROWS, D), jnp.float32),
        mesh=mesh,
        compiler_params=pltpu.CompilerParams(needs_layout_passes=False),
        scratch_types=[
            pltpu.VMEM((2, IB, 2, G), jnp.int32),
            pltpu.VMEM((2, G), jnp.int32),
            pltpu.VMEM((2, G, D), jnp.float32),
            pltpu.VMEM((RING,), jnp.int32),
            pltpu.VMEM((RING,), jnp.int32),
            pltpu.VMEM((16, D), jnp.float32),
            pltpu.VMEM_SHARED((SP_ROWS, D), jnp.float32),
            pltpu.SemaphoreType.DMA,
            pltpu.SemaphoreType.DMA,
            pltpu.SemaphoreType.DMA,
            pltpu.SemaphoreType.DMA,
            pltpu.SemaphoreType.DMA,
        ],
    )(_sc_body)
    return k(whall2d, tcat)


def _epi_body(a_ref, n_ref, o_ref):
    o_ref[...] = jnp.maximum(a_ref[...] * n_ref[...], 0.0)


def _epilogue(acc, norm):
    return pl.pallas_call(
        _epi_body,
        grid=(N // BN, NDIV),
        in_specs=[
            pl.BlockSpec((BN, D), lambda i, d: (d * (N // BN) + i, 0)),
            pl.BlockSpec((BN, 1), lambda i, d: (i, 0)),
        ],
        out_specs=pl.BlockSpec((BN, D), lambda i, d: (i, d)),
        out_shape=jax.ShapeDtypeStruct((N, NDIV * D), jnp.float32),
    )(acc, norm)


def kernel(feature, edge_index, subgraph_idx, norm, W):
    pad = E_PAD - E
    src_p = jnp.pad(edge_index[0], (0, pad)).reshape(E_PAD // G, G)
    dst_p = jnp.pad(edge_index[1], (0, pad)).reshape(E_PAD // G, G)
    div_p = jnp.pad(subgraph_idx, (0, pad)).reshape(E_PAD // G, G)

    whall = _whall(feature, norm, W).reshape(ROWS, D)
    tcat = _edge_indices(src_p, dst_p, div_p)
    acc = _sc_scatter(whall, tcat)
    return _epilogue(acc, norm)


# speedup vs baseline: 10.6236x; 1.0124x over previous
import functools

import jax
import jax.numpy as jnp
from jax import lax
from jax.experimental import pallas as pl
from jax.experimental.pallas import tpu as pltpu
from jax.experimental.pallas import tpu_sc as plsc

N = 10000
E = 320000
D = 128
NDIV = 9
ROWS = NDIV * N

NC = 2
NS = 16
G = 128
E_PAD = 327680
EPT = E_PAD // NS
NG = EPT // G
CH = 11264
NPASS = 4
SP_ROWS = CH + 128
ACC_ROWS = NC * NPASS * CH
PAD_DST = ROWS

BN = 400


def _whall_body(f_ref, n_ref, w_ref, o_ref):
    w = w_ref[0]
    acc = lax.dot_general(f_ref[...], w, (((1,), (1,)), ((), ())),
                          preferred_element_type=jnp.float32)
    o_ref[0] = acc * n_ref[...]


def _whall(feature, norm, W):
    return pl.pallas_call(
        _whall_body,
        grid=(NDIV, N // BN),
        in_specs=[
            pl.BlockSpec((BN, D), lambda d, i: (i, 0)),
            pl.BlockSpec((BN, 1), lambda d, i: (i, 0)),
            pl.BlockSpec((1, D, D), lambda d, i: (d, 0, 0)),
        ],
        out_specs=pl.BlockSpec((1, BN, D), lambda d, i: (d, i, 0)),
        out_shape=jax.ShapeDtypeStruct((NDIV, N, D), jnp.float32),
    )(feature, norm, W)


def _idx_body(src_ref, dst_ref, div_ref, o_ref):
    i = pl.program_id(0)
    r = lax.broadcasted_iota(jnp.int32, src_ref.shape, 0)
    c = lax.broadcasted_iota(jnp.int32, src_ref.shape, 1)
    pos = (i * src_ref.shape[0] + r) * G + c
    valid = pos < E
    div = div_ref[...]
    o_ref[:, 0, :] = jnp.where(valid, div * N + src_ref[...], 0)
    o_ref[:, 1, :] = jnp.where(valid, div * N + dst_ref[...], PAD_DST)


def _edge_indices(src_p, dst_p, div_p):
    NGRP = E_PAD // G
    RB = 64
    spec = pl.BlockSpec((RB, G), lambda i: (i, 0))
    return pl.pallas_call(
        _idx_body,
        grid=(NGRP // RB,),
        in_specs=[spec, spec, spec],
        out_specs=pl.BlockSpec((RB, 2, G), lambda i: (i, 0, 0)),
        out_shape=jax.ShapeDtypeStruct((NGRP, 2, G), jnp.int32),
    )(src_p, dst_p, div_p)


RING = 4 * G
IB = 4
NB = NG // IB


def _sc_body(whall_hbm, tcat_hbm, acc_hbm,
             idx_v, sidx2, rows_v, gstage, sstage, zbuf, spacc,
             sem_i0, sem_i1, sem_r0, sem_r1, sem_z):
    c = lax.axis_index("c")
    s = lax.axis_index("s")
    sem_i = (sem_i0, sem_i1)
    sem_r = (sem_r0, sem_r1)

    zv = jnp.zeros((16,), jnp.float32)

    def _zrow(r, carry):
        for k in range(8):
            zbuf[r, pl.ds(16 * k, 16)] = zv
        return carry

    lax.fori_loop(0, zbuf.shape[0], _zrow, 0)

    ZR = zbuf.shape[0]
    PT = SP_ROWS // NS
    WT = CH // NS

    def _consume(j):
        poff = pl.multiple_of((j & 3) * G, G)
        for pp in range(2):
            @pl.when((j & 1) == pp)
            def _():
                for k in range(8):
                    sidx2[pp, pl.ds(16 * k, 16)] = \
                        sstage[pl.ds(poff + 16 * k, 16)]
                for h in range(2):
                    pltpu.make_async_copy(
                        whall_hbm.at[gstage.at[pl.ds(0, G // 2)]],
                        rows_v.at[pp].at[pl.ds(h * (G // 2), G // 2)],
                        sem_r[pp]).wait()
                pltpu.sync_copy(rows_v.at[pp], spacc.at[sidx2.at[pp]],
                                add=True)

    def _fire(j):
        off = pl.multiple_of((j & 3) * G, G)
        for pp in range(2):
            @pl.when((j & 1) == pp)
            def _():
                for h in range(2):
                    pltpu.make_async_copy(
                        whall_hbm.at[gstage.at[pl.ds(off + h * (G // 2),
                                                     G // 2)]],
                        rows_v.at[pp].at[pl.ds(h * (G // 2), G // 2)],
                        sem_r[pp]).start()

        @pl.when(j >= 1)
        def _():
            _consume(j - 1)

    for p in range(NPASS):
        base = (NC * p + c) * CH

        z0 = s * PT
        for j in range(PT // ZR):
            pltpu.make_async_copy(zbuf.at[pl.ds(0, ZR)],
                                  spacc.at[pl.ds(z0 + j * ZR, ZR)],
                                  sem_z).start()
        if PT % ZR:
            pltpu.make_async_copy(zbuf.at[pl.ds(0, PT % ZR)],
                                  spacc.at[pl.ds(z0 + (PT // ZR) * ZR,
                                                 PT % ZR)],
                                  sem_z).start()
        for j in range(PT // ZR):
            pltpu.make_async_copy(zbuf.at[pl.ds(0, ZR)],
                                  spacc.at[pl.ds(z0 + j * ZR, ZR)],
                                  sem_z).wait()
        if PT % ZR:
            pltpu.make_async_copy(zbuf.at[pl.ds(0, PT % ZR)],
                                  spacc.at[pl.ds(z0 + (PT // ZR) * ZR,
                                                 PT % ZR)],
                                  sem_z).wait()
        plsc.subcore_barrier()

        pltpu.make_async_copy(tcat_hbm.at[pl.ds(s * NG, IB)], idx_v.at[0],
                              sem_i[0]).start()
        pltpu.make_async_copy(tcat_hbm.at[pl.ds(s * NG + IB, IB)],
                              idx_v.at[1], sem_i[1]).start()

        def _outer(o, carry):
            cntv, nf = carry
            for b in range(2):
                blk = o * 2 + b
                pltpu.make_async_copy(tcat_hbm.at[pl.ds(s * NG + blk * IB,
                                                        IB)],
                                      idx_v.at[b], sem_i[b]).wait()
                for j in range(IB):
                    for k in range(8):
                        ts = idx_v[b, j, 0, pl.ds(16 * k, 16)]
                        td = idx_v[b, j, 1, pl.ds(16 * k, 16)]
                        loc = td - base
                        ok = (loc >= 0) & (loc < CH)
                        oki = ok.astype(jnp.int32)
                        pos = (cntv + plsc.cumsum(oki) - 1) & (RING - 1)
                        plsc.store_scatter(gstage, [pos], ts, mask=ok)
                        plsc.store_scatter(sstage, [pos], loc, mask=ok)
                        cntv = cntv + plsc.all_reduce_population_count(ok)
                    fire_cond = (jnp.max(cntv) - nf * G) >= G

                    @pl.when(fire_cond)
                    def _():
                        _fire(nf)
                    nf = jnp.where(fire_cond, nf + 1, nf)
                @pl.when(blk + 2 < NB)
                def _():
                    pltpu.make_async_copy(
                        tcat_hbm.at[pl.ds(s * NG + (blk + 2) * IB, IB)],
                        idx_v.at[b], sem_i[b]).start()
            return (cntv, nf)

        cntv, nf = lax.fori_loop(0, NB // 2, _outer,
                                 (jnp.zeros((16,), jnp.int32), jnp.int32(0)))
        cnt = jnp.max(cntv)

        lane = lax.broadcasted_iota(jnp.int32, (16,), 0)
        zsrc = jnp.zeros((16,), jnp.int32)
        zdst = jnp.full((16,), CH, jnp.int32)
        for k in range(8):
            pos = (cnt + 16 * k + lane) & (RING - 1)
            plsc.store_scatter(gstage, [pos], zsrc)
            plsc.store_scatter(sstage, [pos], zdst)
        nb_end = (cnt + G - 1) >> 7
        for _extra in range(2):
            fire_cond = nf < nb_end

            @pl.when(fire_cond)
            def _():
                _fire(nf)
            nf = jnp.where(fire_cond, nf + 1, nf)

        @pl.when(nf >= 1)
        def _():
            _consume(nf - 1)

        plsc.subcore_barrier()

        w0 = s * WT
        pltpu.sync_copy(spacc.at[pl.ds(w0, WT)],
                        acc_hbm.at[pl.ds(base + w0, WT)])
        plsc.subcore_barrier()


def _sc_scatter(whall2d, tcat):
    mesh = plsc.VectorSubcoreMesh(core_axis_name="c", subcore_axis_name="s",
                                  num_cores=NC, num_subcores=NS)
    k = functools.partial(
        pl.kernel,
        out_type=jax.ShapeDtypeStruct((ACC_ROWS, D), jnp.float32),
        mesh=mesh,
        compiler_params=pltpu.CompilerParams(needs_layout_passes=False),
        scratch_types=[
            pltpu.VMEM((2, IB, 2, G), jnp.int32),
            pltpu.VMEM((2, G), jnp.int32),
            pltpu.VMEM((2, G, D), jnp.float32),
            pltpu.VMEM((RING,), jnp.int32),
            pltpu.VMEM((RING,), jnp.int32),
            pltpu.VMEM((16, D), jnp.float32),
            pltpu.VMEM_SHARED((SP_ROWS, D), jnp.float32),
            pltpu.SemaphoreType.DMA,
            pltpu.SemaphoreType.DMA,
            pltpu.SemaphoreType.DMA,
            pltpu.SemaphoreType.DMA,
            pltpu.SemaphoreType.DMA,
        ],
    )(_sc_body)
    return k(whall2d, tcat)


def _epi_body(a_ref, n_ref, o_ref):
    o_ref[...] = jnp.maximum(a_ref[...] * n_ref[...], 0.0)


def _epilogue(acc, norm):
    return pl.pallas_call(
        _epi_body,
        grid=(N // BN, NDIV),
        in_specs=[
            pl.BlockSpec((BN, D), lambda i, d: (d * (N // BN) + i, 0)),
            pl.BlockSpec((BN, 1), lambda i, d: (i, 0)),
        ],
        out_specs=pl.BlockSpec((BN, D), lambda i, d: (i, d)),
        out_shape=jax.ShapeDtypeStruct((N, NDIV * D), jnp.float32),
    )(acc, norm)


def kernel(feature, edge_index, subgraph_idx, norm, W):
    pad = E_PAD - E
    src_p = jnp.pad(edge_index[0], (0, pad)).reshape(E_PAD // G, G)
    dst_p = jnp.pad(edge_index[1], (0, pad)).reshape(E_PAD // G, G)
    div_p = jnp.pad(subgraph_idx, (0, pad)).reshape(E_PAD // G, G)

    whall = _whall(feature, norm, W).reshape(ROWS, D)
    tcat = _edge_indices(src_p, dst_p, div_p)
    acc = _sc_scatter(whall, tcat)
    return _epilogue(acc, norm)


# speedup vs baseline: 12.3132x; 1.1590x over previous
import functools

import jax
import jax.numpy as jnp
from jax import lax
from jax.experimental import pallas as pl
from jax.experimental.pallas import tpu as pltpu
from jax.experimental.pallas import tpu_sc as plsc

N = 10000
E = 320000
D = 128
NDIV = 9
ROWS = NDIV * N

NC = 2
NS = 16
G = 128
E_PAD = 327680
EPT = E_PAD // NS
NG = EPT // G
CH = 11264
NPASS = 4
SP_ROWS = CH + 128
ACC_ROWS = NC * NPASS * CH
PAD_DST = ROWS

BN = 400


def _whall_body(f_ref, n_ref, w_ref, o_ref):
    w = w_ref[0]
    acc = lax.dot_general(f_ref[...], w, (((1,), (1,)), ((), ())),
                          preferred_element_type=jnp.float32)
    o_ref[0] = acc * n_ref[...]


def _whall(feature, norm, W):
    return pl.pallas_call(
        _whall_body,
        grid=(NDIV, N // BN),
        in_specs=[
            pl.BlockSpec((BN, D), lambda d, i: (i, 0)),
            pl.BlockSpec((BN, 1), lambda d, i: (i, 0)),
            pl.BlockSpec((1, D, D), lambda d, i: (d, 0, 0)),
        ],
        out_specs=pl.BlockSpec((1, BN, D), lambda d, i: (d, i, 0)),
        out_shape=jax.ShapeDtypeStruct((NDIV, N, D), jnp.float32),
    )(feature, norm, W)


def _idx_body(src_ref, dst_ref, div_ref, o_ref):
    i = pl.program_id(0)
    r = lax.broadcasted_iota(jnp.int32, src_ref.shape, 0)
    c = lax.broadcasted_iota(jnp.int32, src_ref.shape, 1)
    pos = (i * src_ref.shape[0] + r) * G + c
    valid = pos < E
    div = div_ref[...]
    o_ref[:, 0, :] = jnp.where(valid, div * N + src_ref[...], 0)
    o_ref[:, 1, :] = jnp.where(valid, div * N + dst_ref[...], PAD_DST)


def _edge_indices(src_p, dst_p, div_p):
    NGRP = E_PAD // G
    RB = 64
    spec = pl.BlockSpec((RB, G), lambda i: (i, 0))
    return pl.pallas_call(
        _idx_body,
        grid=(NGRP // RB,),
        in_specs=[spec, spec, spec],
        out_specs=pl.BlockSpec((RB, 2, G), lambda i: (i, 0, 0)),
        out_shape=jax.ShapeDtypeStruct((NGRP, 2, G), jnp.int32),
    )(src_p, dst_p, div_p)


RING = 8 * G
IB = 4
NB = NG // IB
GF = 64
NBUF = 4


def _sc_body(whall_hbm, tcat_hbm, acc_hbm,
             idx_v, sidx2, rows_v, gstage, sstage, zbuf, spacc,
             sem_i0, sem_i1, sem_r0, sem_r1, sem_r2, sem_r3, sem_z):
    c = lax.axis_index("c")
    s = lax.axis_index("s")
    sem_i = (sem_i0, sem_i1)
    sem_r = (sem_r0, sem_r1, sem_r2, sem_r3)

    zv = jnp.zeros((16,), jnp.float32)

    def _zrow(r, carry):
        for k in range(8):
            zbuf[r, pl.ds(16 * k, 16)] = zv
        return carry

    lax.fori_loop(0, zbuf.shape[0], _zrow, 0)

    ZR = zbuf.shape[0]
    PT = SP_ROWS // NS
    WT = CH // NS

    def _consume(j):
        poff = pl.multiple_of((j & (RING // GF - 1)) * GF, GF)
        for pp in range(NBUF):
            @pl.when((j & (NBUF - 1)) == pp)
            def _():
                for k in range(GF // 16):
                    sidx2[pp, pl.ds(16 * k, 16)] = \
                        sstage[pl.ds(poff + 16 * k, 16)]
                pltpu.make_async_copy(whall_hbm.at[gstage.at[pl.ds(0, GF)]],
                                      rows_v.at[pp], sem_r[pp]).wait()
                pltpu.sync_copy(rows_v.at[pp], spacc.at[sidx2.at[pp]],
                                add=True)

    def _fire(j):
        off = pl.multiple_of((j & (RING // GF - 1)) * GF, GF)
        for pp in range(NBUF):
            @pl.when((j & (NBUF - 1)) == pp)
            def _():
                pltpu.make_async_copy(whall_hbm.at[gstage.at[pl.ds(off, GF)]],
                                      rows_v.at[pp], sem_r[pp]).start()

        @pl.when(j >= NBUF - 1)
        def _():
            _consume(j - (NBUF - 1))

    def _fire_next(nf_):
        _fire(nf_)
        return nf_ + 1

    for p in range(NPASS):
        base = (NC * p + c) * CH

        z0 = s * PT
        for j in range(PT // ZR):
            pltpu.make_async_copy(zbuf.at[pl.ds(0, ZR)],
                                  spacc.at[pl.ds(z0 + j * ZR, ZR)],
                                  sem_z).start()
        if PT % ZR:
            pltpu.make_async_copy(zbuf.at[pl.ds(0, PT % ZR)],
                                  spacc.at[pl.ds(z0 + (PT // ZR) * ZR,
                                                 PT % ZR)],
                                  sem_z).start()
        for j in range(PT // ZR):
            pltpu.make_async_copy(zbuf.at[pl.ds(0, ZR)],
                                  spacc.at[pl.ds(z0 + j * ZR, ZR)],
                                  sem_z).wait()
        if PT % ZR:
            pltpu.make_async_copy(zbuf.at[pl.ds(0, PT % ZR)],
                                  spacc.at[pl.ds(z0 + (PT // ZR) * ZR,
                                                 PT % ZR)],
                                  sem_z).wait()
        plsc.subcore_barrier()

        pltpu.make_async_copy(tcat_hbm.at[pl.ds(s * NG, IB)], idx_v.at[0],
                              sem_i[0]).start()
        pltpu.make_async_copy(tcat_hbm.at[pl.ds(s * NG + IB, IB)],
                              idx_v.at[1], sem_i[1]).start()

        def _outer(o, carry):
            cntv, nf = carry
            for b in range(2):
                blk = o * 2 + b
                pltpu.make_async_copy(tcat_hbm.at[pl.ds(s * NG + blk * IB,
                                                        IB)],
                                      idx_v.at[b], sem_i[b]).wait()
                for j in range(IB):
                    for k in range(8):
                        ts = idx_v[b, j, 0, pl.ds(16 * k, 16)]
                        td = idx_v[b, j, 1, pl.ds(16 * k, 16)]
                        loc = td - base
                        ok = (loc >= 0) & (loc < CH)
                        oki = ok.astype(jnp.int32)
                        pos = (cntv + plsc.cumsum(oki) - 1) & (RING - 1)
                        plsc.store_scatter(gstage, [pos], ts, mask=ok)
                        plsc.store_scatter(sstage, [pos], loc, mask=ok)
                        cntv = cntv + plsc.all_reduce_population_count(ok)
                cnt_g = jnp.max(cntv)
                nf = lax.while_loop(
                    lambda nf_: (cnt_g - nf_ * GF) >= GF, _fire_next, nf)
                @pl.when(blk + 2 < NB)
                def _():
                    pltpu.make_async_copy(
                        tcat_hbm.at[pl.ds(s * NG + (blk + 2) * IB, IB)],
                        idx_v.at[b], sem_i[b]).start()
            return (cntv, nf)

        cntv, nf = lax.fori_loop(0, NB // 2, _outer,
                                 (jnp.zeros((16,), jnp.int32), jnp.int32(0)))
        cnt = jnp.max(cntv)

        lane = lax.broadcasted_iota(jnp.int32, (16,), 0)
        zsrc = jnp.zeros((16,), jnp.int32)
        zdst = jnp.full((16,), CH, jnp.int32)
        for k in range(GF // 16):
            pos = (cnt + 16 * k + lane) & (RING - 1)
            plsc.store_scatter(gstage, [pos], zsrc)
            plsc.store_scatter(sstage, [pos], zdst)
        nb_end = (cnt + GF - 1) >> 6
        nf = lax.while_loop(lambda nf_: nf_ < nb_end, _fire_next, nf)

        for q in range(NBUF - 1):
            jj = nf - (NBUF - 1) + q

            @pl.when(jj >= 0)
            def _():
                _consume(jj)

        plsc.subcore_barrier()

        w0 = s * WT
        pltpu.sync_copy(spacc.at[pl.ds(w0, WT)],
                        acc_hbm.at[pl.ds(base + w0, WT)])
        plsc.subcore_barrier()


def _sc_scatter(whall2d, tcat):
    mesh = plsc.VectorSubcoreMesh(core_axis_name="c", subcore_axis_name="s",
                                  num_cores=NC, num_subcores=NS)
    k = functools.partial(
        pl.kernel,
        out_type=jax.ShapeDtypeStruct((ACC_ROWS, D), jnp.float32),
        mesh=mesh,
        compiler_params=pltpu.CompilerParams(needs_layout_passes=False),
        scratch_types=[
            pltpu.VMEM((2, IB, 2, G), jnp.int32),
            pltpu.VMEM((NBUF, GF), jnp.int32),
            pltpu.VMEM((NBUF, GF, D), jnp.float32),
            pltpu.VMEM((RING,), jnp.int32),
            pltpu.VMEM((RING,), jnp.int32),
            pltpu.VMEM((16, D), jnp.float32),
            pltpu.VMEM_SHARED((SP_ROWS, D), jnp.float32),
            pltpu.SemaphoreType.DMA,
            pltpu.SemaphoreType.DMA,
            pltpu.SemaphoreType.DMA,
            pltpu.SemaphoreType.DMA,
            pltpu.SemaphoreType.DMA,
            pltpu.SemaphoreType.DMA,
            pltpu.SemaphoreType.DMA,
        ],
    )(_sc_body)
    return k(whall2d, tcat)


def _epi_body(a_ref, n_ref, o_ref):
    o_ref[...] = jnp.maximum(a_ref[...] * n_ref[...], 0.0)


def _epilogue(acc, norm):
    return pl.pallas_call(
        _epi_body,
        grid=(N // BN, NDIV),
        in_specs=[
            pl.BlockSpec((BN, D), lambda i, d: (d * (N // BN) + i, 0)),
            pl.BlockSpec((BN, 1), lambda i, d: (i, 0)),
        ],
        out_specs=pl.BlockSpec((BN, D), lambda i, d: (i, d)),
        out_shape=jax.ShapeDtypeStruct((N, NDIV * D), jnp.float32),
    )(acc, norm)


def kernel(feature, edge_index, subgraph_idx, norm, W):
    pad = E_PAD - E
    src_p = jnp.pad(edge_index[0], (0, pad)).reshape(E_PAD // G, G)
    dst_p = jnp.pad(edge_index[1], (0, pad)).reshape(E_PAD // G, G)
    div_p = jnp.pad(subgraph_idx, (0, pad)).reshape(E_PAD // G, G)

    whall = _whall(feature, norm, W).reshape(ROWS, D)
    tcat = _edge_indices(src_p, dst_p, div_p)
    acc = _sc_scatter(whall, tcat)
    return _epilogue(acc, norm)


# speedup vs baseline: 12.7044x; 1.0318x over previous
import functools

import jax
import jax.numpy as jnp
from jax import lax
from jax.experimental import pallas as pl
from jax.experimental.pallas import tpu as pltpu
from jax.experimental.pallas import tpu_sc as plsc

N = 10000
E = 320000
D = 128
NDIV = 9
ROWS = NDIV * N

NC = 2
NS = 16
G = 128
E_PAD = 327680
EPT = E_PAD // NS
NG = EPT // G
CH = 11264
NPASS = 4
SP_ROWS = CH + 128
ACC_ROWS = NC * NPASS * CH
PAD_DST = ROWS

BN = 400


def _whall_body(f_ref, n_ref, w_ref, o_ref):
    w = w_ref[0]
    acc = lax.dot_general(f_ref[...], w, (((1,), (1,)), ((), ())),
                          preferred_element_type=jnp.float32)
    o_ref[0] = acc * n_ref[...]


def _whall(feature, norm, W):
    return pl.pallas_call(
        _whall_body,
        grid=(NDIV, N // BN),
        in_specs=[
            pl.BlockSpec((BN, D), lambda d, i: (i, 0)),
            pl.BlockSpec((BN, 1), lambda d, i: (i, 0)),
            pl.BlockSpec((1, D, D), lambda d, i: (d, 0, 0)),
        ],
        out_specs=pl.BlockSpec((1, BN, D), lambda d, i: (d, i, 0)),
        out_shape=jax.ShapeDtypeStruct((NDIV, N, D), jnp.float32),
    )(feature, norm, W)


def _idx_body(src_ref, dst_ref, div_ref, o_ref):
    i = pl.program_id(0)
    r = lax.broadcasted_iota(jnp.int32, src_ref.shape, 0)
    c = lax.broadcasted_iota(jnp.int32, src_ref.shape, 1)
    pos = (i * src_ref.shape[0] + r) * G + c
    valid = pos < E
    div = div_ref[...]
    o_ref[:, 0, :] = jnp.where(valid, div * N + src_ref[...], 0)
    o_ref[:, 1, :] = jnp.where(valid, div * N + dst_ref[...], PAD_DST)


def _edge_indices(src_p, dst_p, div_p):
    NGRP = E_PAD // G
    RB = 64
    spec = pl.BlockSpec((RB, G), lambda i: (i, 0))
    return pl.pallas_call(
        _idx_body,
        grid=(NGRP // RB,),
        in_specs=[spec, spec, spec],
        out_specs=pl.BlockSpec((RB, 2, G), lambda i: (i, 0, 0)),
        out_shape=jax.ShapeDtypeStruct((NGRP, 2, G), jnp.int32),
    )(src_p, dst_p, div_p)


RING = 8 * G
IB = 4
NB = NG // IB
GF = 32
NBUF = 8


def _sc_body(whall_hbm, tcat_hbm, acc_hbm,
             idx_v, sidx2, rows_v, gstage, sstage, zbuf, spacc,
             sem_i0, sem_i1, sem_r0, sem_r1, sem_r2, sem_r3,
             sem_r4, sem_r5, sem_r6, sem_r7, sem_z):
    c = lax.axis_index("c")
    s = lax.axis_index("s")
    sem_i = (sem_i0, sem_i1)
    sem_r = (sem_r0, sem_r1, sem_r2, sem_r3, sem_r4, sem_r5, sem_r6, sem_r7)

    zv = jnp.zeros((16,), jnp.float32)

    def _zrow(r, carry):
        for k in range(8):
            zbuf[r, pl.ds(16 * k, 16)] = zv
        return carry

    lax.fori_loop(0, zbuf.shape[0], _zrow, 0)

    ZR = zbuf.shape[0]
    PT = SP_ROWS // NS
    WT = CH // NS

    def _consume(j):
        poff = pl.multiple_of((j & (RING // GF - 1)) * GF, GF)
        for pp in range(NBUF):
            @pl.when((j & (NBUF - 1)) == pp)
            def _():
                for k in range(GF // 16):
                    sidx2[pp, pl.ds(16 * k, 16)] = \
                        sstage[pl.ds(poff + 16 * k, 16)]
                pltpu.make_async_copy(whall_hbm.at[gstage.at[pl.ds(0, GF)]],
                                      rows_v.at[pp], sem_r[pp]).wait()
                pltpu.sync_copy(rows_v.at[pp], spacc.at[sidx2.at[pp]],
                                add=True)

    def _fire(j):
        off = pl.multiple_of((j & (RING // GF - 1)) * GF, GF)
        for pp in range(NBUF):
            @pl.when((j & (NBUF - 1)) == pp)
            def _():
                pltpu.make_async_copy(whall_hbm.at[gstage.at[pl.ds(off, GF)]],
                                      rows_v.at[pp], sem_r[pp]).start()

        @pl.when(j >= NBUF - 1)
        def _():
            _consume(j - (NBUF - 1))

    def _fire_next(nf_):
        _fire(nf_)
        return nf_ + 1

    for p in range(NPASS):
        base = (NC * p + c) * CH

        z0 = s * PT
        for j in range(PT // ZR):
            pltpu.make_async_copy(zbuf.at[pl.ds(0, ZR)],
                                  spacc.at[pl.ds(z0 + j * ZR, ZR)],
                                  sem_z).start()
        if PT % ZR:
            pltpu.make_async_copy(zbuf.at[pl.ds(0, PT % ZR)],
                                  spacc.at[pl.ds(z0 + (PT // ZR) * ZR,
                                                 PT % ZR)],
                                  sem_z).start()
        for j in range(PT // ZR):
            pltpu.make_async_copy(zbuf.at[pl.ds(0, ZR)],
                                  spacc.at[pl.ds(z0 + j * ZR, ZR)],
                                  sem_z).wait()
        if PT % ZR:
            pltpu.make_async_copy(zbuf.at[pl.ds(0, PT % ZR)],
                                  spacc.at[pl.ds(z0 + (PT // ZR) * ZR,
                                                 PT % ZR)],
                                  sem_z).wait()
        plsc.subcore_barrier()

        pltpu.make_async_copy(tcat_hbm.at[pl.ds(s * NG, IB)], idx_v.at[0],
                              sem_i[0]).start()
        pltpu.make_async_copy(tcat_hbm.at[pl.ds(s * NG + IB, IB)],
                              idx_v.at[1], sem_i[1]).start()

        def _outer(o, carry):
            cntv, nf = carry
            for b in range(2):
                blk = o * 2 + b
                pltpu.make_async_copy(tcat_hbm.at[pl.ds(s * NG + blk * IB,
                                                        IB)],
                                      idx_v.at[b], sem_i[b]).wait()
                for j in range(IB):
                    for k in range(8):
                        ts = idx_v[b, j, 0, pl.ds(16 * k, 16)]
                        td = idx_v[b, j, 1, pl.ds(16 * k, 16)]
                        loc = td - base
                        ok = (loc >= 0) & (loc < CH)
                        oki = ok.astype(jnp.int32)
                        pos = (cntv + plsc.cumsum(oki) - 1) & (RING - 1)
                        plsc.store_scatter(gstage, [pos], ts, mask=ok)
                        plsc.store_scatter(sstage, [pos], loc, mask=ok)
                        cntv = cntv + plsc.all_reduce_population_count(ok)
                cnt_g = jnp.max(cntv)
                nf = lax.while_loop(
                    lambda nf_: (cnt_g - nf_ * GF) >= GF, _fire_next, nf)
                @pl.when(blk + 2 < NB)
                def _():
                    pltpu.make_async_copy(
                        tcat_hbm.at[pl.ds(s * NG + (blk + 2) * IB, IB)],
                        idx_v.at[b], sem_i[b]).start()
            return (cntv, nf)

        cntv, nf = lax.fori_loop(0, NB // 2, _outer,
                                 (jnp.zeros((16,), jnp.int32), jnp.int32(0)))
        cnt = jnp.max(cntv)

        lane = lax.broadcasted_iota(jnp.int32, (16,), 0)
        zsrc = jnp.zeros((16,), jnp.int32)
        zdst = jnp.full((16,), CH, jnp.int32)
        for k in range(GF // 16):
            pos = (cnt + 16 * k + lane) & (RING - 1)
            plsc.store_scatter(gstage, [pos], zsrc)
            plsc.store_scatter(sstage, [pos], zdst)
        nb_end = (cnt + GF - 1) >> 5
        nf = lax.while_loop(lambda nf_: nf_ < nb_end, _fire_next, nf)

        def _drain(jc):
            _consume(jc)
            return jc + 1

        lax.while_loop(lambda jc: jc < nf, _drain,
                       jnp.maximum(nf - (NBUF - 1), 0))

        plsc.subcore_barrier()

        w0 = s * WT
        pltpu.sync_copy(spacc.at[pl.ds(w0, WT)],
                        acc_hbm.at[pl.ds(base + w0, WT)])
        plsc.subcore_barrier()


def _sc_scatter(whall2d, tcat):
    mesh = plsc.VectorSubcoreMesh(core_axis_name="c", subcore_axis_name="s",
                                  num_cores=NC, num_subcores=NS)
    k = functools.partial(
        pl.kernel,
        out_type=jax.ShapeDtypeStruct((ACC_ROWS, D), jnp.float32),
        mesh=mesh,
        compiler_params=pltpu.CompilerParams(needs_layout_passes=False),
        scratch_types=[
            pltpu.VMEM((2, IB, 2, G), jnp.int32),
            pltpu.VMEM((NBUF, GF), jnp.int32),
            pltpu.VMEM((NBUF, GF, D), jnp.float32),
            pltpu.VMEM((RING,), jnp.int32),
            pltpu.VMEM((RING,), jnp.int32),
            pltpu.VMEM((16, D), jnp.float32),
            pltpu.VMEM_SHARED((SP_ROWS, D), jnp.float32),
            pltpu.SemaphoreType.DMA,
            pltpu.SemaphoreType.DMA,
            pltpu.SemaphoreType.DMA,
            pltpu.SemaphoreType.DMA,
            pltpu.SemaphoreType.DMA,
            pltpu.SemaphoreType.DMA,
            pltpu.SemaphoreType.DMA,
            pltpu.SemaphoreType.DMA,
            pltpu.SemaphoreType.DMA,
            pltpu.SemaphoreType.DMA,
            pltpu.SemaphoreType.DMA,
        ],
    )(_sc_body)
    return k(whall2d, tcat)


def _epi_body(a_ref, n_ref, o_ref):
    o_ref[...] = jnp.maximum(a_ref[...] * n_ref[...], 0.0)


def _epilogue(acc, norm):
    return pl.pallas_call(
        _epi_body,
        grid=(N // BN, NDIV),
        in_specs=[
            pl.BlockSpec((BN, D), lambda i, d: (d * (N // BN) + i, 0)),
            pl.BlockSpec((BN, 1), lambda i, d: (i, 0)),
        ],
        out_specs=pl.BlockSpec((BN, D), lambda i, d: (i, d)),
        out_shape=jax.ShapeDtypeStruct((N, NDIV * D), jnp.float32),
    )(acc, norm)


def kernel(feature, edge_index, subgraph_idx, norm, W):
    pad = E_PAD - E
    src_p = jnp.pad(edge_index[0], (0, pad)).reshape(E_PAD // G, G)
    dst_p = jnp.pad(edge_index[1], (0, pad)).reshape(E_PAD // G, G)
    div_p = jnp.pad(subgraph_idx, (0, pad)).reshape(E_PAD // G, G)

    whall = _whall(feature, norm, W).reshape(ROWS, D)
    tcat = _edge_indices(src_p, dst_p, div_p)
    acc = _sc_scatter(whall, tcat)
    return _epilogue(acc, norm)
